# Initial kernel scaffold; baseline (speedup 1.0000x reference)
import jax
import jax.numpy as jnp
from jax.experimental import pallas as pl


def kernel(center1, feat1, center2, feat2, Wf0, bf0, gf0, bef0, Ws0, bs0, gs0, bes0, W1, b1, g1, be1):
    raise NotImplementedError("write your pallas kernel here")


# speedup vs baseline: 18.2661x; 18.2661x over previous
import jax
import jax.numpy as jnp
from jax import lax
from jax.experimental import pallas as pl
from jax.experimental.pallas import tpu as pltpu

B, N1, N2, C1, C2, M0, M1 = 4, 4096, 1024, 32, 64, 64, 64
TILE = 1024
NT = N1 // TILE
EPS = 1e-5
BIG = 1e30


def _dot_bf(a, b):
    return lax.dot_general(a.astype(jnp.bfloat16), b.astype(jnp.bfloat16),
                           (((1,), (0,)), ((), ())),
                           preferred_element_type=jnp.float32)


def _dot_hi(a, b):
    return lax.dot_general(a, b, (((1,), (0,)), ((), ())),
                           preferred_element_type=jnp.float32,
                           precision=lax.Precision.HIGHEST)


def _dot_t_bf(a, b):
    return lax.dot_general(a.astype(jnp.bfloat16), b.astype(jnp.bfloat16),
                           (((0,), (0,)), ((), ())),
                           preferred_element_type=jnp.float32)


def _norms(c):
    return (c[0:1] * c[0:1] + c[1:2] * c[1:2]) + c[2:3] * c[2:3]


def _prep_kernel(feat2_ref, feat1_ref, Wf0_ref, bf0_ref, gf0_ref, bef0_ref,
                 Ws0_ref, bs0_ref, gs0_ref, bes0_ref, p2_ref, skip_ref):
    Wf0 = Wf0_ref[...]
    for b in range(B):
        p2_ref[b] = _dot_t_bf(feat2_ref[b], Wf0) + bf0_ref[...]
    y2 = p2_ref[...].reshape(B * N2, M0)
    m2 = jnp.mean(y2, axis=0, keepdims=True)
    v2 = jnp.mean((y2 - m2) ** 2, axis=0, keepdims=True)
    p2n = gf0_ref[...] * (y2 - m2) / jnp.sqrt(v2 + EPS) + bef0_ref[...]
    p2_ref[...] = p2n.reshape(B, N2, M0)
    Ws0 = Ws0_ref[...]
    for b in range(B):
        skip_ref[b] = _dot_t_bf(feat1_ref[b], Ws0) + bs0_ref[...]
    y1 = skip_ref[...].reshape(B * N1, M0)
    m1 = jnp.mean(y1, axis=0, keepdims=True)
    v1 = jnp.mean((y1 - m1) ** 2, axis=0, keepdims=True)
    sk = gs0_ref[...] * (y1 - m1) / jnp.sqrt(v1 + EPS) + bes0_ref[...]
    skip_ref[...] = sk.reshape(B, N1, M0)


def _search_kernel(c1_ref, c2_ref, p2_ref, skip_ref, W1_ref, b1_ref,
                   z_ref, s1_ref, s2_ref):
    c1 = c1_ref[0]
    c2 = c2_ref[0]
    g = lax.dot_general(c1.astype(jnp.bfloat16), c2.astype(jnp.bfloat16),
                        (((0,), (0,)), ((), ())),
                        preferred_element_type=jnp.float32)
    n1c = _norms(c1).T
    n2 = _norms(c2)
    d = (n1c + n2) - 2.0 * g
    iota = lax.broadcasted_iota(jnp.int32, (TILE, N2), 1)
    m1 = jnp.min(d, axis=1, keepdims=True)
    i1 = jnp.min(jnp.where(d == m1, iota, N2), axis=1, keepdims=True)
    d = jnp.where(iota == i1, BIG, d)
    m2 = jnp.min(d, axis=1, keepdims=True)
    i2 = jnp.min(jnp.where(d == m2, iota, N2), axis=1, keepdims=True)
    d = jnp.where(iota == i2, BIG, d)
    m3 = jnp.min(d, axis=1, keepdims=True)
    i3 = jnp.min(jnp.where(d == m3, iota, N2), axis=1, keepdims=True)
    r1 = 1.0 / (m1 + 1e-8)
    r2 = 1.0 / (m2 + 1e-8)
    r3 = 1.0 / (m3 + 1e-8)
    s = r1 + r2 + r3
    zero = jnp.zeros((), jnp.float32)
    wm = (jnp.where(iota == i1, r1 / s, zero)
          + jnp.where(iota == i2, r2 / s, zero)
          + jnp.where(iota == i3, r3 / s, zero))
    interp = _dot_hi(wm, p2_ref[0])
    x = jnp.maximum(interp + skip_ref[0], 0.0)
    z = _dot_bf(x, W1_ref[...]) + b1_ref[...]
    z_ref[0] = z
    zs = jnp.sum(z, axis=0, keepdims=True)
    zs2 = jnp.sum(z * z, axis=0, keepdims=True)

    first = (pl.program_id(0) == 0) & (pl.program_id(1) == 0)

    @pl.when(first)
    def _():
        s1_ref[...] = jnp.zeros((8, M1), jnp.float32)
        s2_ref[...] = jnp.zeros((8, M1), jnp.float32)

    s1_ref[...] += jnp.broadcast_to(zs, (8, M1))
    s2_ref[...] += jnp.broadcast_to(zs2, (8, M1))


def _final_kernel(z_ref, s1_ref, s2_ref, g1_ref, be1_ref, out_ref):
    n = jnp.float32(B * N1)
    m = s1_ref[0:1, :] / n
    v = s2_ref[0:1, :] / n - m * m
    scale = g1_ref[...] * lax.rsqrt(v + EPS)
    shift = be1_ref[...] - m * scale
    y = jnp.maximum(z_ref[0] * scale + shift, 0.0)
    out_ref[0] = y.T


def kernel(center1, feat1, center2, feat2, Wf0, bf0, gf0, bef0,
           Ws0, bs0, gs0, bes0, W1, b1, g1, be1):
    r = lambda p: p.reshape(1, -1)
    p2, skip = pl.pallas_call(
        _prep_kernel,
        out_shape=[
            jax.ShapeDtypeStruct((B, N2, M0), jnp.float32),
            jax.ShapeDtypeStruct((B, N1, M0), jnp.float32),
        ],
    )(feat2, feat1, Wf0, r(bf0), r(gf0), r(bef0), Ws0, r(bs0), r(gs0), r(bes0))

    z, s1, s2 = pl.pallas_call(
        _search_kernel,
        grid=(B, NT),
        in_specs=[
            pl.BlockSpec((1, 3, TILE), lambda b, t: (b, 0, t)),
            pl.BlockSpec((1, 3, N2), lambda b, t: (b, 0, 0)),
            pl.BlockSpec((1, N2, M0), lambda b, t: (b, 0, 0)),
            pl.BlockSpec((1, TILE, M0), lambda b, t: (b, t, 0)),
            pl.BlockSpec((M0, M1), lambda b, t: (0, 0)),
            pl.BlockSpec((1, M1), lambda b, t: (0, 0)),
        ],
        out_specs=[
            pl.BlockSpec((1, TILE, M1), lambda b, t: (b, t, 0)),
            pl.BlockSpec((8, M1), lambda b, t: (0, 0)),
            pl.BlockSpec((8, M1), lambda b, t: (0, 0)),
        ],
        out_shape=[
            jax.ShapeDtypeStruct((B, N1, M1), jnp.float32),
            jax.ShapeDtypeStruct((8, M1), jnp.float32),
            jax.ShapeDtypeStruct((8, M1), jnp.float32),
        ],
    )(center1, center2, p2, skip, W1, r(b1))

    out = pl.pallas_call(
        _final_kernel,
        grid=(B, NT),
        in_specs=[
            pl.BlockSpec((1, TILE, M1), lambda b, t: (b, t, 0)),
            pl.BlockSpec((8, M1), lambda b, t: (0, 0)),
            pl.BlockSpec((8, M1), lambda b, t: (0, 0)),
            pl.BlockSpec((1, M1), lambda b, t: (0, 0)),
            pl.BlockSpec((1, M1), lambda b, t: (0, 0)),
        ],
        out_specs=pl.BlockSpec((1, M1, TILE), lambda b, t: (b, 0, t)),
        out_shape=jax.ShapeDtypeStruct((B, M1, N1), jnp.float32),
    )(z, s1, s2, r(g1), r(be1))
    return out


# speedup vs baseline: 20.8972x; 1.1440x over previous
import functools

import jax
import jax.numpy as jnp
from jax import lax
from jax.experimental import pallas as pl
from jax.experimental.pallas import tpu as pltpu
from jax.experimental.pallas import tpu_sc as plsc

B, N1, N2, C1, C2, M0, M1 = 4, 4096, 1024, 32, 64, 64, 64
TILE = 1024
NT = N1 // TILE
EPS = 1e-5
BIG = 1e30

NROWS = 3 * B * N1
SC_WORKERS = 32
SC_CHUNK = 128
SC_CPW = NROWS // (SC_WORKERS * SC_CHUNK)


def _dot_bf(a, b):
    return lax.dot_general(a.astype(jnp.bfloat16), b.astype(jnp.bfloat16),
                           (((1,), (0,)), ((), ())),
                           preferred_element_type=jnp.float32)


def _dot_t_bf(a, b):
    return lax.dot_general(a.astype(jnp.bfloat16), b.astype(jnp.bfloat16),
                           (((0,), (0,)), ((), ())),
                           preferred_element_type=jnp.float32)


def _norms(c):
    return (c[0:1] * c[0:1] + c[1:2] * c[1:2]) + c[2:3] * c[2:3]


def _prep_kernel(feat2_ref, feat1_ref, Wf0_ref, bf0_ref, gf0_ref, bef0_ref,
                 Ws0_ref, bs0_ref, gs0_ref, bes0_ref, p2_ref, skip_ref):
    Wf0 = Wf0_ref[...]
    for b in range(B):
        p2_ref[pl.ds(b * N2, N2)] = _dot_t_bf(feat2_ref[b], Wf0) + bf0_ref[...]
    y2 = p2_ref[...]
    m2 = jnp.mean(y2, axis=0, keepdims=True)
    v2 = jnp.mean((y2 - m2) ** 2, axis=0, keepdims=True)
    p2_ref[...] = gf0_ref[...] * (y2 - m2) / jnp.sqrt(v2 + EPS) + bef0_ref[...]
    Ws0 = Ws0_ref[...]
    for b in range(B):
        skip_ref[b] = _dot_t_bf(feat1_ref[b], Ws0) + bs0_ref[...]
    y1 = skip_ref[...].reshape(B * N1, M0)
    m1 = jnp.mean(y1, axis=0, keepdims=True)
    v1 = jnp.mean((y1 - m1) ** 2, axis=0, keepdims=True)
    sk = gs0_ref[...] * (y1 - m1) / jnp.sqrt(v1 + EPS) + bes0_ref[...]
    skip_ref[...] = sk.reshape(B, N1, M0)


def _search_kernel(c1_ref, c2_ref, aux_ref):
    c1 = c1_ref[0]
    c2 = c2_ref[0]
    g = lax.dot_general(c1.astype(jnp.bfloat16), c2.astype(jnp.bfloat16),
                        (((0,), (0,)), ((), ())),
                        preferred_element_type=jnp.float32)
    n1c = _norms(c1).T
    n2 = _norms(c2)
    d = (n1c + n2) - 2.0 * g
    iota = lax.broadcasted_iota(jnp.int32, (TILE, N2), 1)
    m1 = jnp.min(d, axis=1, keepdims=True)
    i1 = jnp.min(jnp.where(d == m1, iota, N2), axis=1, keepdims=True)
    d = jnp.where(iota == i1, BIG, d)
    m2 = jnp.min(d, axis=1, keepdims=True)
    i2 = jnp.min(jnp.where(d == m2, iota, N2), axis=1, keepdims=True)
    d = jnp.where(iota == i2, BIG, d)
    m3 = jnp.min(d, axis=1, keepdims=True)
    i3 = jnp.min(jnp.where(d == m3, iota, N2), axis=1, keepdims=True)
    r1 = 1.0 / (m1 + 1e-8)
    r2 = 1.0 / (m2 + 1e-8)
    r3 = 1.0 / (m3 + 1e-8)
    s = r1 + r2 + r3
    boff = pl.program_id(0) * N2
    packed = jnp.concatenate(
        [(i1 + boff).astype(jnp.float32), (i2 + boff).astype(jnp.float32),
         (i3 + boff).astype(jnp.float32), r1 / s, r2 / s, r3 / s, r1, r1],
        axis=1)
    aux_ref[0] = packed.T


def _sc_gather_kernel(aux_hbm, p2_hbm, out_hbm, p2_sp, idxf_v, idx_v, rows_v, sem0, sem1):
    sid = lax.axis_index("s")
    wid = sid * 2 + lax.axis_index("c")

    @pl.when(sid == 0)
    def _():
        pltpu.sync_copy(p2_hbm, p2_sp)

    plsc.subcore_barrier()
    for j in range(SC_CPW):
        s = wid * SC_CPW + j
        k = s // (NROWS // SC_CHUNK // 3)
        ss = s % (NROWS // SC_CHUNK // 3)
        blk = ss // 8
        lane = (ss % 8) * SC_CHUNK
        pltpu.sync_copy(aux_hbm.at[blk, k, pl.ds(lane, SC_CHUNK)], idxf_v.at[j])
    for j in range(SC_CPW):
        for i in range(SC_CHUNK // 16):
            idx_v[j, pl.ds(i * 16, 16)] = idxf_v[j, pl.ds(i * 16, 16)].astype(jnp.int32)
    sems = [sem0, sem1]
    copies = [None, None]
    copies[0] = pltpu.async_copy(p2_sp.at[idx_v.at[0]], rows_v.at[0], sems[0])
    for j in range(SC_CPW):
        if j + 1 < SC_CPW:
            copies[(j + 1) % 2] = pltpu.async_copy(
                p2_sp.at[idx_v.at[j + 1]], rows_v.at[(j + 1) % 2], sems[(j + 1) % 2])
        copies[j % 2].wait()
        s = wid * SC_CPW + j
        pltpu.sync_copy(rows_v.at[j % 2], out_hbm.at[pl.ds(s * SC_CHUNK, SC_CHUNK)])


def _combine_kernel(gat_ref, aux_ref, skip_ref, W1_ref, b1_ref,
                    z_ref, s1_ref, s2_ref):
    auxt = aux_ref[0].T
    w1 = auxt[:, 3:4]
    w2 = auxt[:, 4:5]
    w3 = auxt[:, 5:6]
    interp = w1 * gat_ref[0, 0] + w2 * gat_ref[1, 0] + w3 * gat_ref[2, 0]
    x = jnp.maximum(interp + skip_ref[0], 0.0)
    z = _dot_bf(x, W1_ref[...]) + b1_ref[...]
    z_ref[0] = z
    zs = jnp.sum(z, axis=0, keepdims=True)
    zs2 = jnp.sum(z * z, axis=0, keepdims=True)

    first = (pl.program_id(0) == 0) & (pl.program_id(1) == 0)

    @pl.when(first)
    def _():
        s1_ref[...] = jnp.zeros((8, M1), jnp.float32)
        s2_ref[...] = jnp.zeros((8, M1), jnp.float32)

    s1_ref[...] += jnp.broadcast_to(zs, (8, M1))
    s2_ref[...] += jnp.broadcast_to(zs2, (8, M1))


def _final_kernel(z_ref, s1_ref, s2_ref, g1_ref, be1_ref, out_ref):
    n = jnp.float32(B * N1)
    m = s1_ref[0:1, :] / n
    v = s2_ref[0:1, :] / n - m * m
    scale = g1_ref[...] * lax.rsqrt(v + EPS)
    shift = be1_ref[...] - m * scale
    y = jnp.maximum(z_ref[0] * scale + shift, 0.0)
    out_ref[0] = y.T


def kernel(center1, feat1, center2, feat2, Wf0, bf0, gf0, bef0,
           Ws0, bs0, gs0, bes0, W1, b1, g1, be1):
    r = lambda p: p.reshape(1, -1)
    p2, skip = pl.pallas_call(
        _prep_kernel,
        out_shape=[
            jax.ShapeDtypeStruct((B * N2, M0), jnp.float32),
            jax.ShapeDtypeStruct((B, N1, M0), jnp.float32),
        ],
    )(feat2, feat1, Wf0, r(bf0), r(gf0), r(bef0), Ws0, r(bs0), r(gs0), r(bes0))

    aux = pl.pallas_call(
        _search_kernel,
        grid=(B, NT),
        in_specs=[
            pl.BlockSpec((1, 3, TILE), lambda b, t: (b, 0, t)),
            pl.BlockSpec((1, 3, N2), lambda b, t: (b, 0, 0)),
        ],
        out_specs=pl.BlockSpec((1, 8, TILE), lambda b, t: (b * NT + t, 0, 0)),
        out_shape=jax.ShapeDtypeStruct((B * NT, 8, TILE), jnp.float32),
    )(center1, center2)

    sc_gather = functools.partial(
        pl.kernel,
        mesh=plsc.VectorSubcoreMesh(core_axis_name="c", subcore_axis_name="s"),
        out_type=jax.ShapeDtypeStruct((NROWS, M0), jnp.float32),
        scratch_types=[
            pltpu.VMEM_SHARED((B * N2, M0), jnp.float32),
            pltpu.VMEM((SC_CPW, SC_CHUNK), jnp.float32),
            pltpu.VMEM((SC_CPW, SC_CHUNK), jnp.int32),
            pltpu.VMEM((2, SC_CHUNK, M0), jnp.float32),
            pltpu.SemaphoreType.DMA,
            pltpu.SemaphoreType.DMA,
        ],
    )(_sc_gather_kernel)
    gathered = sc_gather(aux, p2)

    z, s1, s2 = pl.pallas_call(
        _combine_kernel,
        grid=(B, NT),
        in_specs=[
            pl.BlockSpec((3, 1, TILE, M0), lambda b, t: (0, b * NT + t, 0, 0)),
            pl.BlockSpec((1, 8, TILE), lambda b, t: (b * NT + t, 0, 0)),
            pl.BlockSpec((1, TILE, M0), lambda b, t: (b, t, 0)),
            pl.BlockSpec((M0, M1), lambda b, t: (0, 0)),
            pl.BlockSpec((1, M1), lambda b, t: (0, 0)),
        ],
        out_specs=[
            pl.BlockSpec((1, TILE, M1), lambda b, t: (b, t, 0)),
            pl.BlockSpec((8, M1), lambda b, t: (0, 0)),
            pl.BlockSpec((8, M1), lambda b, t: (0, 0)),
        ],
        out_shape=[
            jax.ShapeDtypeStruct((B, N1, M1), jnp.float32),
            jax.ShapeDtypeStruct((8, M1), jnp.float32),
            jax.ShapeDtypeStruct((8, M1), jnp.float32),
        ],
    )(gathered.reshape(3, B * NT, TILE, M0), aux, skip, W1, r(b1))

    out = pl.pallas_call(
        _final_kernel,
        grid=(B, NT),
        in_specs=[
            pl.BlockSpec((1, TILE, M1), lambda b, t: (b, t, 0)),
            pl.BlockSpec((8, M1), lambda b, t: (0, 0)),
            pl.BlockSpec((8, M1), lambda b, t: (0, 0)),
            pl.BlockSpec((1, M1), lambda b, t: (0, 0)),
            pl.BlockSpec((1, M1), lambda b, t: (0, 0)),
        ],
        out_specs=pl.BlockSpec((1, M1, TILE), lambda b, t: (b, 0, t)),
        out_shape=jax.ShapeDtypeStruct((B, M1, N1), jnp.float32),
    )(z, s1, s2, r(g1), r(be1))
    return out


# speedup vs baseline: 23.0527x; 1.1031x over previous
import functools

import jax
import jax.numpy as jnp
from jax import lax
from jax.experimental import pallas as pl
from jax.experimental.pallas import tpu as pltpu
from jax.experimental.pallas import tpu_sc as plsc

B, N1, N2, C1, C2, M0, M1 = 4, 4096, 1024, 32, 64, 64, 64
TILE = 1024
NT = N1 // TILE
EPS = 1e-5
BIG = 1e30

NROWS = 3 * B * N1
SC_WORKERS = 32
SC_CHUNK = 128
SC_CPW = NROWS // (SC_WORKERS * SC_CHUNK)


def _dot_bf(a, b):
    return lax.dot_general(a.astype(jnp.bfloat16), b.astype(jnp.bfloat16),
                           (((1,), (0,)), ((), ())),
                           preferred_element_type=jnp.float32)


def _dot_t_bf(a, b):
    return lax.dot_general(a.astype(jnp.bfloat16), b.astype(jnp.bfloat16),
                           (((0,), (0,)), ((), ())),
                           preferred_element_type=jnp.float32)


def _norms(c):
    return (c[0:1] * c[0:1] + c[1:2] * c[1:2]) + c[2:3] * c[2:3]


def _bn_rows(y, n, g, b):
    m = jnp.sum(y, axis=0, keepdims=True) / n
    ym = y - m
    v = jnp.sum(ym * ym, axis=0, keepdims=True) / n
    scale = g * lax.rsqrt(v + EPS)
    return ym * scale + b


def _prep_kernel(feat2_ref, feat1_ref, Wf0_ref, bf0_ref, gf0_ref, bef0_ref,
                 Ws0_ref, bs0_ref, gs0_ref, bes0_ref, p2_ref, skip_ref):
    f2 = jnp.concatenate([feat2_ref[b] for b in range(B)], axis=1)
    y2 = _dot_t_bf(f2, Wf0_ref[...]) + bf0_ref[...]
    p2_ref[...] = _bn_rows(y2, B * N2, gf0_ref[...], bef0_ref[...])
    f1 = jnp.concatenate([feat1_ref[b] for b in range(B)], axis=1)
    y1 = _dot_t_bf(f1, Ws0_ref[...]) + bs0_ref[...]
    sk = _bn_rows(y1, B * N1, gs0_ref[...], bes0_ref[...])
    skip_ref[...] = sk.reshape(B, N1, M0)


def _search_kernel(c1_ref, c2_ref, aux_ref):
    c1 = c1_ref[0]
    c2 = c2_ref[0]
    g = lax.dot_general(c1.astype(jnp.bfloat16), c2.astype(jnp.bfloat16),
                        (((0,), (0,)), ((), ())),
                        preferred_element_type=jnp.float32)
    n1c = _norms(c1).T
    n2 = _norms(c2)
    d = (n1c + n2) - 2.0 * g
    iota = lax.broadcasted_iota(jnp.int32, (1, N2), 1).astype(jnp.float32)
    nf = float(N2)
    m1 = jnp.min(d, axis=1, keepdims=True)
    i1 = jnp.min(jnp.where(d == m1, iota, nf), axis=1, keepdims=True)
    d = jnp.where(iota == i1, BIG, d)
    m2 = jnp.min(d, axis=1, keepdims=True)
    i2 = jnp.min(jnp.where(d == m2, iota, nf), axis=1, keepdims=True)
    d = jnp.where(iota == i2, BIG, d)
    m3 = jnp.min(d, axis=1, keepdims=True)
    i3 = jnp.min(jnp.where(d == m3, iota, nf), axis=1, keepdims=True)
    r1 = 1.0 / (m1 + 1e-8)
    r2 = 1.0 / (m2 + 1e-8)
    r3 = 1.0 / (m3 + 1e-8)
    s = r1 + r2 + r3
    boff = (pl.program_id(0) * N2).astype(jnp.float32)
    packed = jnp.concatenate(
        [i1 + boff, i2 + boff, i3 + boff, r1 / s, r2 / s, r3 / s, r1, r1],
        axis=1)
    aux_ref[0] = packed.T


def _sc_gather_kernel(aux_hbm, p2_hbm, out_hbm, p2_sp, idxf_v, idx_v, rows_v, sem0, sem1):
    sid = lax.axis_index("s")
    wid = sid * 2 + lax.axis_index("c")

    @pl.when(sid == 0)
    def _():
        pltpu.sync_copy(p2_hbm, p2_sp)

    plsc.subcore_barrier()
    for j in range(SC_CPW):
        s = wid * SC_CPW + j
        k = s // (NROWS // SC_CHUNK // 3)
        ss = s % (NROWS // SC_CHUNK // 3)
        blk = ss // 8
        lane = (ss % 8) * SC_CHUNK
        pltpu.sync_copy(aux_hbm.at[blk, k, pl.ds(lane, SC_CHUNK)], idxf_v.at[j])
    for j in range(SC_CPW):
        for i in range(SC_CHUNK // 16):
            idx_v[j, pl.ds(i * 16, 16)] = idxf_v[j, pl.ds(i * 16, 16)].astype(jnp.int32)
    sems = [sem0, sem1]
    copies = [None, None]
    copies[0] = pltpu.async_copy(p2_sp.at[idx_v.at[0]], rows_v.at[0], sems[0])
    for j in range(SC_CPW):
        if j + 1 < SC_CPW:
            copies[(j + 1) % 2] = pltpu.async_copy(
                p2_sp.at[idx_v.at[j + 1]], rows_v.at[(j + 1) % 2], sems[(j + 1) % 2])
        copies[j % 2].wait()
        s = wid * SC_CPW + j
        pltpu.sync_copy(rows_v.at[j % 2], out_hbm.at[pl.ds(s * SC_CHUNK, SC_CHUNK)])


def _combine_kernel(gat_ref, aux_ref, skip_ref, W1_ref, b1_ref,
                    z_ref, s1_ref, s2_ref):
    auxt = aux_ref[0].T
    w1 = auxt[:, 3:4]
    w2 = auxt[:, 4:5]
    w3 = auxt[:, 5:6]
    interp = w1 * gat_ref[0, 0] + w2 * gat_ref[1, 0] + w3 * gat_ref[2, 0]
    x = jnp.maximum(interp + skip_ref[0], 0.0)
    z = _dot_bf(x, W1_ref[...]) + b1_ref[...]
    z_ref[0] = z
    zs = jnp.sum(z, axis=0, keepdims=True)
    zs2 = jnp.sum(z * z, axis=0, keepdims=True)

    first = (pl.program_id(0) == 0) & (pl.program_id(1) == 0)

    @pl.when(first)
    def _():
        s1_ref[...] = jnp.zeros((8, M1), jnp.float32)
        s2_ref[...] = jnp.zeros((8, M1), jnp.float32)

    s1_ref[...] += jnp.broadcast_to(zs, (8, M1))
    s2_ref[...] += jnp.broadcast_to(zs2, (8, M1))


def _final_kernel(z_ref, s1_ref, s2_ref, g1_ref, be1_ref, out_ref):
    n = jnp.float32(B * N1)
    m = s1_ref[0:1, :] / n
    v = s2_ref[0:1, :] / n - m * m
    scale = g1_ref[...] * lax.rsqrt(v + EPS)
    shift = be1_ref[...] - m * scale
    y = jnp.maximum(z_ref[0] * scale + shift, 0.0)
    out_ref[0] = y.T


def kernel(center1, feat1, center2, feat2, Wf0, bf0, gf0, bef0,
           Ws0, bs0, gs0, bes0, W1, b1, g1, be1):
    r = lambda p: p.reshape(1, -1)
    p2, skip = pl.pallas_call(
        _prep_kernel,
        out_shape=[
            jax.ShapeDtypeStruct((B * N2, M0), jnp.float32),
            jax.ShapeDtypeStruct((B, N1, M0), jnp.float32),
        ],
    )(feat2, feat1, Wf0, r(bf0), r(gf0), r(bef0), Ws0, r(bs0), r(gs0), r(bes0))

    aux = pl.pallas_call(
        _search_kernel,
        grid=(B, NT),
        in_specs=[
            pl.BlockSpec((1, 3, TILE), lambda b, t: (b, 0, t)),
            pl.BlockSpec((1, 3, N2), lambda b, t: (b, 0, 0)),
        ],
        out_specs=pl.BlockSpec((1, 8, TILE), lambda b, t: (b * NT + t, 0, 0)),
        out_shape=jax.ShapeDtypeStruct((B * NT, 8, TILE), jnp.float32),
    )(center1, center2)

    sc_gather = functools.partial(
        pl.kernel,
        mesh=plsc.VectorSubcoreMesh(core_axis_name="c", subcore_axis_name="s"),
        out_type=jax.ShapeDtypeStruct((NROWS, M0), jnp.float32),
        scratch_types=[
            pltpu.VMEM_SHARED((B * N2, M0), jnp.float32),
            pltpu.VMEM((SC_CPW, SC_CHUNK), jnp.float32),
            pltpu.VMEM((SC_CPW, SC_CHUNK), jnp.int32),
            pltpu.VMEM((2, SC_CHUNK, M0), jnp.float32),
            pltpu.SemaphoreType.DMA,
            pltpu.SemaphoreType.DMA,
        ],
    )(_sc_gather_kernel)
    gathered = sc_gather(aux, p2)

    z, s1, s2 = pl.pallas_call(
        _combine_kernel,
        grid=(B, NT),
        in_specs=[
            pl.BlockSpec((3, 1, TILE, M0), lambda b, t: (0, b * NT + t, 0, 0)),
            pl.BlockSpec((1, 8, TILE), lambda b, t: (b * NT + t, 0, 0)),
            pl.BlockSpec((1, TILE, M0), lambda b, t: (b, t, 0)),
            pl.BlockSpec((M0, M1), lambda b, t: (0, 0)),
            pl.BlockSpec((1, M1), lambda b, t: (0, 0)),
        ],
        out_specs=[
            pl.BlockSpec((1, TILE, M1), lambda b, t: (b, t, 0)),
            pl.BlockSpec((8, M1), lambda b, t: (0, 0)),
            pl.BlockSpec((8, M1), lambda b, t: (0, 0)),
        ],
        out_shape=[
            jax.ShapeDtypeStruct((B, N1, M1), jnp.float32),
            jax.ShapeDtypeStruct((8, M1), jnp.float32),
            jax.ShapeDtypeStruct((8, M1), jnp.float32),
        ],
    )(gathered.reshape(3, B * NT, TILE, M0), aux, skip, W1, r(b1))

    out = pl.pallas_call(
        _final_kernel,
        grid=(B, NT),
        in_specs=[
            pl.BlockSpec((1, TILE, M1), lambda b, t: (b, t, 0)),
            pl.BlockSpec((8, M1), lambda b, t: (0, 0)),
            pl.BlockSpec((8, M1), lambda b, t: (0, 0)),
            pl.BlockSpec((1, M1), lambda b, t: (0, 0)),
            pl.BlockSpec((1, M1), lambda b, t: (0, 0)),
        ],
        out_specs=pl.BlockSpec((1, M1, TILE), lambda b, t: (b, 0, t)),
        out_shape=jax.ShapeDtypeStruct((B, M1, N1), jnp.float32),
    )(z, s1, s2, r(g1), r(be1))
    return out


# speedup vs baseline: 24.6355x; 1.0687x over previous
import functools

import jax
import jax.numpy as jnp
from jax import lax
from jax.experimental import pallas as pl
from jax.experimental.pallas import tpu as pltpu
from jax.experimental.pallas import tpu_sc as plsc

B, N1, N2, C1, C2, M0, M1 = 4, 4096, 1024, 32, 64, 64, 64
TILE = 2048
NT = N1 // TILE
EPS = 1e-5
BIG = 1e30

NROWS = 3 * B * N1
SC_WORKERS = 32
SC_CHUNK = 128
SC_CPW = NROWS // (SC_WORKERS * SC_CHUNK)


def _dot_bf(a, b):
    return lax.dot_general(a.astype(jnp.bfloat16), b.astype(jnp.bfloat16),
                           (((1,), (0,)), ((), ())),
                           preferred_element_type=jnp.float32)


def _dot_t_bf(a, b):
    return lax.dot_general(a.astype(jnp.bfloat16), b.astype(jnp.bfloat16),
                           (((0,), (0,)), ((), ())),
                           preferred_element_type=jnp.float32)


def _norms(c):
    return (c[0:1] * c[0:1] + c[1:2] * c[1:2]) + c[2:3] * c[2:3]


def _bn_rows(y, n, g, b):
    m = jnp.sum(y, axis=0, keepdims=True) / n
    ym = y - m
    v = jnp.sum(ym * ym, axis=0, keepdims=True) / n
    scale = g * lax.rsqrt(v + EPS)
    return ym * scale + b


def _prep_kernel(feat2_ref, feat1_ref, Wf0_ref, bf0_ref, gf0_ref, bef0_ref,
                 Ws0_ref, bs0_ref, gs0_ref, bes0_ref, p2_ref, skip_ref):
    f2 = jnp.concatenate([feat2_ref[b] for b in range(B)], axis=1)
    y2 = _dot_t_bf(f2, Wf0_ref[...]) + bf0_ref[...]
    p2_ref[...] = _bn_rows(y2, B * N2, gf0_ref[...], bef0_ref[...])
    f1 = jnp.concatenate([feat1_ref[b] for b in range(B)], axis=1)
    y1 = _dot_t_bf(f1, Ws0_ref[...]) + bs0_ref[...]
    sk = _bn_rows(y1, B * N1, gs0_ref[...], bes0_ref[...])
    skip_ref[...] = sk.reshape(B, N1, M0)


def _search_kernel(c1_ref, c2_ref, aux_ref):
    c1 = c1_ref[0]
    c2 = c2_ref[0]
    g = lax.dot_general(c1.astype(jnp.bfloat16), c2.astype(jnp.bfloat16),
                        (((0,), (0,)), ((), ())),
                        preferred_element_type=jnp.float32)
    n1c = _norms(c1).T
    n2 = _norms(c2)
    d = (n1c + n2) - 2.0 * g
    iota = lax.broadcasted_iota(jnp.int32, (1, N2), 1).astype(jnp.float32)
    nf = float(N2)
    m1 = jnp.min(d, axis=1, keepdims=True)
    i1 = jnp.min(jnp.where(d == m1, iota, nf), axis=1, keepdims=True)
    d = jnp.where(iota == i1, BIG, d)
    m2 = jnp.min(d, axis=1, keepdims=True)
    i2 = jnp.min(jnp.where(d == m2, iota, nf), axis=1, keepdims=True)
    d = jnp.where(iota == i2, BIG, d)
    m3 = jnp.min(d, axis=1, keepdims=True)
    i3 = jnp.min(jnp.where(d == m3, iota, nf), axis=1, keepdims=True)
    r1 = 1.0 / (m1 + 1e-8)
    r2 = 1.0 / (m2 + 1e-8)
    r3 = 1.0 / (m3 + 1e-8)
    s = r1 + r2 + r3
    boff = (pl.program_id(0) * N2).astype(jnp.float32)
    packed = jnp.concatenate(
        [i1 + boff, i2 + boff, i3 + boff, r1 / s, r2 / s, r3 / s, r1, r1],
        axis=1)
    aux_ref[0] = packed.T


def _sc_gather_kernel(aux_hbm, p2_hbm, out_hbm, p2_sp, idxf_v, idx_v, rows_v, sem0, sem1):
    sid = lax.axis_index("s")
    wid = sid * 2 + lax.axis_index("c")

    @pl.when(sid == 0)
    def _():
        pltpu.sync_copy(p2_hbm, p2_sp)

    plsc.subcore_barrier()
    for j in range(SC_CPW):
        s = wid * SC_CPW + j
        k = s // (NROWS // SC_CHUNK // 3)
        ss = s % (NROWS // SC_CHUNK // 3)
        pb = TILE // SC_CHUNK
        blk = ss // pb
        lane = (ss % pb) * SC_CHUNK
        pltpu.sync_copy(aux_hbm.at[blk, k, pl.ds(lane, SC_CHUNK)], idxf_v.at[j])
    for j in range(SC_CPW):
        for i in range(SC_CHUNK // 16):
            idx_v[j, pl.ds(i * 16, 16)] = idxf_v[j, pl.ds(i * 16, 16)].astype(jnp.int32)
    sems = [sem0, sem1]
    copies = [None, None]
    copies[0] = pltpu.async_copy(p2_sp.at[idx_v.at[0]], rows_v.at[0], sems[0])
    for j in range(SC_CPW):
        if j + 1 < SC_CPW:
            copies[(j + 1) % 2] = pltpu.async_copy(
                p2_sp.at[idx_v.at[j + 1]], rows_v.at[(j + 1) % 2], sems[(j + 1) % 2])
        copies[j % 2].wait()
        s = wid * SC_CPW + j
        pltpu.sync_copy(rows_v.at[j % 2], out_hbm.at[pl.ds(s * SC_CHUNK, SC_CHUNK)])


def _combine_kernel(gat_ref, aux_ref, skip_ref, W1_ref, b1_ref,
                    z_ref, s1_ref, s2_ref):
    auxt = aux_ref[0].T
    w1 = auxt[:, 3:4]
    w2 = auxt[:, 4:5]
    w3 = auxt[:, 5:6]
    interp = w1 * gat_ref[0, 0] + w2 * gat_ref[1, 0] + w3 * gat_ref[2, 0]
    x = jnp.maximum(interp + skip_ref[0], 0.0)
    z = _dot_bf(x, W1_ref[...]) + b1_ref[...]
    z_ref[0] = z
    zs = jnp.sum(z, axis=0, keepdims=True)
    zs2 = jnp.sum(z * z, axis=0, keepdims=True)

    first = (pl.program_id(0) == 0) & (pl.program_id(1) == 0)

    @pl.when(first)
    def _():
        s1_ref[...] = jnp.zeros((8, M1), jnp.float32)
        s2_ref[...] = jnp.zeros((8, M1), jnp.float32)

    s1_ref[...] += jnp.broadcast_to(zs, (8, M1))
    s2_ref[...] += jnp.broadcast_to(zs2, (8, M1))


def _final_kernel(z_ref, s1_ref, s2_ref, g1_ref, be1_ref, out_ref):
    n = jnp.float32(B * N1)
    m = s1_ref[0:1, :] / n
    v = s2_ref[0:1, :] / n - m * m
    scale = g1_ref[...] * lax.rsqrt(v + EPS)
    shift = be1_ref[...] - m * scale
    y = jnp.maximum(z_ref[0] * scale + shift, 0.0)
    out_ref[0] = y.T


def kernel(center1, feat1, center2, feat2, Wf0, bf0, gf0, bef0,
           Ws0, bs0, gs0, bes0, W1, b1, g1, be1):
    r = lambda p: p.reshape(1, -1)
    p2, skip = pl.pallas_call(
        _prep_kernel,
        out_shape=[
            jax.ShapeDtypeStruct((B * N2, M0), jnp.float32),
            jax.ShapeDtypeStruct((B, N1, M0), jnp.float32),
        ],
    )(feat2, feat1, Wf0, r(bf0), r(gf0), r(bef0), Ws0, r(bs0), r(gs0), r(bes0))

    aux = pl.pallas_call(
        _search_kernel,
        grid=(B, NT),
        in_specs=[
            pl.BlockSpec((1, 3, TILE), lambda b, t: (b, 0, t)),
            pl.BlockSpec((1, 3, N2), lambda b, t: (b, 0, 0)),
        ],
        out_specs=pl.BlockSpec((1, 8, TILE), lambda b, t: (b * NT + t, 0, 0)),
        out_shape=jax.ShapeDtypeStruct((B * NT, 8, TILE), jnp.float32),
    )(center1, center2)

    sc_gather = functools.partial(
        pl.kernel,
        mesh=plsc.VectorSubcoreMesh(core_axis_name="c", subcore_axis_name="s"),
        out_type=jax.ShapeDtypeStruct((NROWS, M0), jnp.float32),
        scratch_types=[
            pltpu.VMEM_SHARED((B * N2, M0), jnp.float32),
            pltpu.VMEM((SC_CPW, SC_CHUNK), jnp.float32),
            pltpu.VMEM((SC_CPW, SC_CHUNK), jnp.int32),
            pltpu.VMEM((2, SC_CHUNK, M0), jnp.float32),
            pltpu.SemaphoreType.DMA,
            pltpu.SemaphoreType.DMA,
        ],
    )(_sc_gather_kernel)
    gathered = sc_gather(aux, p2)

    z, s1, s2 = pl.pallas_call(
        _combine_kernel,
        grid=(B, NT),
        in_specs=[
            pl.BlockSpec((3, 1, TILE, M0), lambda b, t: (0, b * NT + t, 0, 0)),
            pl.BlockSpec((1, 8, TILE), lambda b, t: (b * NT + t, 0, 0)),
            pl.BlockSpec((1, TILE, M0), lambda b, t: (b, t, 0)),
            pl.BlockSpec((M0, M1), lambda b, t: (0, 0)),
            pl.BlockSpec((1, M1), lambda b, t: (0, 0)),
        ],
        out_specs=[
            pl.BlockSpec((1, TILE, M1), lambda b, t: (b, t, 0)),
            pl.BlockSpec((8, M1), lambda b, t: (0, 0)),
            pl.BlockSpec((8, M1), lambda b, t: (0, 0)),
        ],
        out_shape=[
            jax.ShapeDtypeStruct((B, N1, M1), jnp.float32),
            jax.ShapeDtypeStruct((8, M1), jnp.float32),
            jax.ShapeDtypeStruct((8, M1), jnp.float32),
        ],
    )(gathered.reshape(3, B * NT, TILE, M0), aux, skip, W1, r(b1))

    out = pl.pallas_call(
        _final_kernel,
        grid=(B, NT),
        in_specs=[
            pl.BlockSpec((1, TILE, M1), lambda b, t: (b, t, 0)),
            pl.BlockSpec((8, M1), lambda b, t: (0, 0)),
            pl.BlockSpec((8, M1), lambda b, t: (0, 0)),
            pl.BlockSpec((1, M1), lambda b, t: (0, 0)),
            pl.BlockSpec((1, M1), lambda b, t: (0, 0)),
        ],
        out_specs=pl.BlockSpec((1, M1, TILE), lambda b, t: (b, 0, t)),
        out_shape=jax.ShapeDtypeStruct((B, M1, N1), jnp.float32),
    )(z, s1, s2, r(g1), r(be1))
    return out


# speedup vs baseline: 25.4035x; 1.0312x over previous
import functools

import jax
import jax.numpy as jnp
from jax import lax
from jax.experimental import pallas as pl
from jax.experimental.pallas import tpu as pltpu
from jax.experimental.pallas import tpu_sc as plsc

B, N1, N2, C1, C2, M0, M1 = 4, 4096, 1024, 32, 64, 64, 64
TILE = 4096
NT = N1 // TILE
EPS = 1e-5
BIG = 1e30

NROWS = 3 * B * N1
SC_WORKERS = 32
SC_CHUNK = 128
SC_CPW = NROWS // (SC_WORKERS * SC_CHUNK)


def _dot_bf(a, b):
    return lax.dot_general(a.astype(jnp.bfloat16), b.astype(jnp.bfloat16),
                           (((1,), (0,)), ((), ())),
                           preferred_element_type=jnp.float32)


def _dot_t_bf(a, b):
    return lax.dot_general(a.astype(jnp.bfloat16), b.astype(jnp.bfloat16),
                           (((0,), (0,)), ((), ())),
                           preferred_element_type=jnp.float32)


def _norms(c):
    return (c[0:1] * c[0:1] + c[1:2] * c[1:2]) + c[2:3] * c[2:3]


def _bn_rows(y, n, g, b):
    m = jnp.sum(y, axis=0, keepdims=True) / n
    ym = y - m
    v = jnp.sum(ym * ym, axis=0, keepdims=True) / n
    scale = g * lax.rsqrt(v + EPS)
    return ym * scale + b


def _prep_kernel(feat2_ref, feat1_ref, Wf0_ref, bf0_ref, gf0_ref, bef0_ref,
                 Ws0_ref, bs0_ref, gs0_ref, bes0_ref, p2_ref, skip_ref):
    f2 = jnp.concatenate([feat2_ref[b] for b in range(B)], axis=1)
    y2 = _dot_t_bf(f2, Wf0_ref[...]) + bf0_ref[...]
    p2_ref[...] = _bn_rows(y2, B * N2, gf0_ref[...], bef0_ref[...])
    f1 = jnp.concatenate([feat1_ref[b] for b in range(B)], axis=1)
    y1 = _dot_t_bf(f1, Ws0_ref[...]) + bs0_ref[...]
    sk = _bn_rows(y1, B * N1, gs0_ref[...], bes0_ref[...])
    skip_ref[...] = sk.reshape(B, N1, M0)


def _search_kernel(c1_ref, c2_ref, aux_ref):
    c1 = c1_ref[0]
    c2 = c2_ref[0]
    g = lax.dot_general(c1.astype(jnp.bfloat16), c2.astype(jnp.bfloat16),
                        (((0,), (0,)), ((), ())),
                        preferred_element_type=jnp.float32)
    n1c = _norms(c1).T
    n2 = _norms(c2)
    d = (n1c + n2) - 2.0 * g
    iota = lax.broadcasted_iota(jnp.int32, (1, N2), 1).astype(jnp.float32)
    nf = float(N2)
    m1 = jnp.min(d, axis=1, keepdims=True)
    i1 = jnp.min(jnp.where(d == m1, iota, nf), axis=1, keepdims=True)
    d = jnp.where(iota == i1, BIG, d)
    m2 = jnp.min(d, axis=1, keepdims=True)
    i2 = jnp.min(jnp.where(d == m2, iota, nf), axis=1, keepdims=True)
    d = jnp.where(iota == i2, BIG, d)
    m3 = jnp.min(d, axis=1, keepdims=True)
    i3 = jnp.min(jnp.where(d == m3, iota, nf), axis=1, keepdims=True)
    r1 = 1.0 / (m1 + 1e-8)
    r2 = 1.0 / (m2 + 1e-8)
    r3 = 1.0 / (m3 + 1e-8)
    s = r1 + r2 + r3
    boff = (pl.program_id(0) * N2).astype(jnp.float32)
    packed = jnp.concatenate(
        [i1 + boff, i2 + boff, i3 + boff, r1 / s, r2 / s, r3 / s, r1, r1],
        axis=1)
    aux_ref[0] = packed.T


def _sc_gather_kernel(aux_hbm, p2_hbm, out_hbm, p2_sp, idxf_v, idx_v, rows_v, sem0, sem1):
    sid = lax.axis_index("s")
    wid = sid * 2 + lax.axis_index("c")

    @pl.when(sid == 0)
    def _():
        pltpu.sync_copy(p2_hbm, p2_sp)

    plsc.subcore_barrier()
    for j in range(SC_CPW):
        s = wid * SC_CPW + j
        k = s // (NROWS // SC_CHUNK // 3)
        ss = s % (NROWS // SC_CHUNK // 3)
        pb = TILE // SC_CHUNK
        blk = ss // pb
        lane = (ss % pb) * SC_CHUNK
        pltpu.sync_copy(aux_hbm.at[blk, k, pl.ds(lane, SC_CHUNK)], idxf_v.at[j])
    for j in range(SC_CPW):
        for i in range(SC_CHUNK // 16):
            idx_v[j, pl.ds(i * 16, 16)] = idxf_v[j, pl.ds(i * 16, 16)].astype(jnp.int32)
    sems = [sem0, sem1]
    copies = [None, None]
    copies[0] = pltpu.async_copy(p2_sp.at[idx_v.at[0]], rows_v.at[0], sems[0])
    for j in range(SC_CPW):
        if j + 1 < SC_CPW:
            copies[(j + 1) % 2] = pltpu.async_copy(
                p2_sp.at[idx_v.at[j + 1]], rows_v.at[(j + 1) % 2], sems[(j + 1) % 2])
        copies[j % 2].wait()
        s = wid * SC_CPW + j
        pltpu.sync_copy(rows_v.at[j % 2], out_hbm.at[pl.ds(s * SC_CHUNK, SC_CHUNK)])


def _combine_kernel(gat_ref, aux_ref, skip_ref, W1_ref, b1_ref,
                    z_ref, s1_ref, s2_ref):
    auxt = aux_ref[0].T
    w1 = auxt[:, 3:4]
    w2 = auxt[:, 4:5]
    w3 = auxt[:, 5:6]
    interp = w1 * gat_ref[0, 0] + w2 * gat_ref[1, 0] + w3 * gat_ref[2, 0]
    x = jnp.maximum(interp + skip_ref[0], 0.0)
    z = _dot_bf(x, W1_ref[...]) + b1_ref[...]
    z_ref[0] = z
    zs = jnp.sum(z, axis=0, keepdims=True)
    zs2 = jnp.sum(z * z, axis=0, keepdims=True)

    first = (pl.program_id(0) == 0) & (pl.program_id(1) == 0)

    @pl.when(first)
    def _():
        s1_ref[...] = jnp.zeros((8, M1), jnp.float32)
        s2_ref[...] = jnp.zeros((8, M1), jnp.float32)

    s1_ref[...] += jnp.broadcast_to(zs, (8, M1))
    s2_ref[...] += jnp.broadcast_to(zs2, (8, M1))


def _final_kernel(z_ref, s1_ref, s2_ref, g1_ref, be1_ref, out_ref):
    n = jnp.float32(B * N1)
    m = s1_ref[0:1, :] / n
    v = s2_ref[0:1, :] / n - m * m
    scale = g1_ref[...] * lax.rsqrt(v + EPS)
    shift = be1_ref[...] - m * scale
    y = jnp.maximum(z_ref[0] * scale + shift, 0.0)
    out_ref[0] = y.T


def kernel(center1, feat1, center2, feat2, Wf0, bf0, gf0, bef0,
           Ws0, bs0, gs0, bes0, W1, b1, g1, be1):
    r = lambda p: p.reshape(1, -1)
    p2, skip = pl.pallas_call(
        _prep_kernel,
        out_shape=[
            jax.ShapeDtypeStruct((B * N2, M0), jnp.float32),
            jax.ShapeDtypeStruct((B, N1, M0), jnp.float32),
        ],
    )(feat2, feat1, Wf0, r(bf0), r(gf0), r(bef0), Ws0, r(bs0), r(gs0), r(bes0))

    aux = pl.pallas_call(
        _search_kernel,
        grid=(B, NT),
        in_specs=[
            pl.BlockSpec((1, 3, TILE), lambda b, t: (b, 0, t)),
            pl.BlockSpec((1, 3, N2), lambda b, t: (b, 0, 0)),
        ],
        out_specs=pl.BlockSpec((1, 8, TILE), lambda b, t: (b * NT + t, 0, 0)),
        out_shape=jax.ShapeDtypeStruct((B * NT, 8, TILE), jnp.float32),
    )(center1, center2)

    sc_gather = functools.partial(
        pl.kernel,
        mesh=plsc.VectorSubcoreMesh(core_axis_name="c", subcore_axis_name="s"),
        out_type=jax.ShapeDtypeStruct((NROWS, M0), jnp.float32),
        scratch_types=[
            pltpu.VMEM_SHARED((B * N2, M0), jnp.float32),
            pltpu.VMEM((SC_CPW, SC_CHUNK), jnp.float32),
            pltpu.VMEM((SC_CPW, SC_CHUNK), jnp.int32),
            pltpu.VMEM((2, SC_CHUNK, M0), jnp.float32),
            pltpu.SemaphoreType.DMA,
            pltpu.SemaphoreType.DMA,
        ],
    )(_sc_gather_kernel)
    gathered = sc_gather(aux, p2)

    z, s1, s2 = pl.pallas_call(
        _combine_kernel,
        grid=(B, NT),
        in_specs=[
            pl.BlockSpec((3, 1, TILE, M0), lambda b, t: (0, b * NT + t, 0, 0)),
            pl.BlockSpec((1, 8, TILE), lambda b, t: (b * NT + t, 0, 0)),
            pl.BlockSpec((1, TILE, M0), lambda b, t: (b, t, 0)),
            pl.BlockSpec((M0, M1), lambda b, t: (0, 0)),
            pl.BlockSpec((1, M1), lambda b, t: (0, 0)),
        ],
        out_specs=[
            pl.BlockSpec((1, TILE, M1), lambda b, t: (b, t, 0)),
            pl.BlockSpec((8, M1), lambda b, t: (0, 0)),
            pl.BlockSpec((8, M1), lambda b, t: (0, 0)),
        ],
        out_shape=[
            jax.ShapeDtypeStruct((B, N1, M1), jnp.float32),
            jax.ShapeDtypeStruct((8, M1), jnp.float32),
            jax.ShapeDtypeStruct((8, M1), jnp.float32),
        ],
    )(gathered.reshape(3, B * NT, TILE, M0), aux, skip, W1, r(b1))

    out = pl.pallas_call(
        _final_kernel,
        grid=(B, NT),
        in_specs=[
            pl.BlockSpec((1, TILE, M1), lambda b, t: (b, t, 0)),
            pl.BlockSpec((8, M1), lambda b, t: (0, 0)),
            pl.BlockSpec((8, M1), lambda b, t: (0, 0)),
            pl.BlockSpec((1, M1), lambda b, t: (0, 0)),
            pl.BlockSpec((1, M1), lambda b, t: (0, 0)),
        ],
        out_specs=pl.BlockSpec((1, M1, TILE), lambda b, t: (b, 0, t)),
        out_shape=jax.ShapeDtypeStruct((B, M1, N1), jnp.float32),
    )(z, s1, s2, r(g1), r(be1))
    return out
